# baseline (speedup 1.0000x reference)
import jax
import jax.numpy as jnp
from jax import lax
from jax.experimental import pallas as pl
from jax.experimental.pallas import tpu as pltpu
from jax.experimental.pallas import tpu_sc as plsc

N_NODES = 10000
D = 128
NC, NS, L = 2, 16, 16
NW = NC * NS
CHUNK = 128
CPT = 80
CPT_BIG = 160
CPT_SMALL = 0
CPP = CPT_BIG + CPT_SMALL
BIG_CORE = 1
EDGES_PAD = NW * CPT * CHUNK
ACC_ROWS = 10112
ROWS_PT = ACC_ROWS // NS
LAST_ROWS = N_NODES - (NS - 1) * ROWS_PT
SAC_ROW = N_NODES


def _zero_vmem(ref, nrows, ncols):
    zv = jnp.zeros((L,), jnp.float32)

    def body(i, carry):
        r = i // (ncols // L)
        c = lax.rem(i, ncols // L)
        ref[r, pl.ds(c * L, L)] = zv
        return carry

    lax.fori_loop(0, nrows * (ncols // L), body, 0)


def _sc_deg_body(dst_hbm, out_hbm, dstb, hist):
    c = lax.axis_index("c")
    s = lax.axis_index("s")
    w = c * NS + s

    pltpu.sync_copy(dst_hbm.at[pl.ds(w * CPT, CPT)], dstb)

    zv = jnp.zeros((L,), jnp.float32)

    def z(i, carry):
        hist[pl.ds(i * L, L)] = zv
        return carry

    lax.fori_loop(0, ACC_ROWS // L, z, 0)

    ov = jnp.ones((L,), jnp.float32)

    def g(i, carry):
        j = i // (CHUNK // L)
        gidx = lax.rem(i, CHUNK // L)
        idx = dstb[j, pl.ds(gidx * L, L)]
        plsc.addupdate_scatter(hist, [idx], ov)
        return carry

    lax.fori_loop(0, CPT * (CHUNK // L), g, 0)

    pltpu.sync_copy(hist, out_hbm.at[w])


_deg_call = pl.kernel(
    _sc_deg_body,
    out_type=jax.ShapeDtypeStruct((NW, ACC_ROWS), jnp.float32),
    mesh=plsc.VectorSubcoreMesh(core_axis_name="c", subcore_axis_name="s"),
    compiler_params=pltpu.CompilerParams(needs_layout_passes=False),
    scratch_types=[
        pltpu.VMEM((CPT, CHUNK), jnp.int32),
        pltpu.VMEM((ACC_ROWS,), jnp.float32),
    ],
)


def _sc_agg_body(h_hbm, src_hbm, dst_hbm, out_hbm, srcb, dring, rows0, rows1,
                 accum, sem0, sem1, semd0, semd1):
    c = lax.axis_index("c")
    s = lax.axis_index("s")

    _zero_vmem(rows0, CHUNK, D)
    for k in range(ROWS_PT // CHUNK):
        pltpu.sync_copy(rows0, accum.at[pl.ds(s * ROWS_PT + k * CHUNK, CHUNK)])
    _rem = ROWS_PT % CHUNK
    if _rem:
        pltpu.sync_copy(
            rows0.at[pl.ds(0, _rem)],
            accum.at[pl.ds(s * ROWS_PT + (ROWS_PT // CHUNK) * CHUNK, _rem)])
    plsc.subcore_barrier()

    def run_chunks(base_row, n_chunks):
        pltpu.sync_copy(src_hbm.at[pl.ds(base_row, n_chunks)],
                        srcb.at[pl.ds(0, n_chunks)])
        pltpu.async_copy(dst_hbm.at[base_row], dring.at[0], semd0)
        pltpu.async_copy(h_hbm.at[srcb.at[0]], rows0, sem0)

        def pair(i, carry):
            j0 = 2 * i
            pltpu.async_copy(dst_hbm.at[base_row + j0 + 1], dring.at[1], semd1)
            pltpu.async_copy(h_hbm.at[srcb.at[j0 + 1]], rows1, sem1)

            pltpu.make_async_copy(dst_hbm.at[base_row + j0], dring.at[0], semd0).wait()
            pltpu.make_async_copy(h_hbm.at[srcb.at[j0]], rows0, sem0).wait()
            pltpu.sync_copy(rows0, accum.at[dring.at[0]], add=True)

            @pl.when(i < n_chunks // 2 - 1)
            def _():
                pltpu.async_copy(dst_hbm.at[base_row + j0 + 2], dring.at[0], semd0)
                pltpu.async_copy(h_hbm.at[srcb.at[j0 + 2]], rows0, sem0)

            pltpu.make_async_copy(dst_hbm.at[base_row + j0 + 1], dring.at[1], semd1).wait()
            pltpu.make_async_copy(h_hbm.at[srcb.at[j0 + 1]], rows1, sem1).wait()
            pltpu.sync_copy(rows1, accum.at[dring.at[1]], add=True)
            return carry

        lax.fori_loop(0, n_chunks // 2, pair, 0)

    @pl.when(c == BIG_CORE)
    def _():
        for half in range(2):
            run_chunks(s * CPP + half * (CPT_BIG // 2), CPT_BIG // 2)

    if CPT_SMALL:
        @pl.when(c != BIG_CORE)
        def _():
            run_chunks(s * CPP + CPT_BIG, CPT_SMALL)

    plsc.subcore_barrier()

    @pl.when(s < NS - 1)
    def _():
        pltpu.sync_copy(accum.at[pl.ds(s * ROWS_PT, ROWS_PT)],
                        out_hbm.at[c, pl.ds(s * ROWS_PT, ROWS_PT)])

    @pl.when(s == NS - 1)
    def _():
        pltpu.sync_copy(accum.at[pl.ds(s * ROWS_PT, LAST_ROWS)],
                        out_hbm.at[c, pl.ds(s * ROWS_PT, LAST_ROWS)])


_agg_call = pl.kernel(
    _sc_agg_body,
    out_type=jax.ShapeDtypeStruct((NC, N_NODES, D), jnp.float32),
    mesh=plsc.VectorSubcoreMesh(core_axis_name="c", subcore_axis_name="s"),
    scratch_types=[
        pltpu.VMEM((max(CPT_BIG // 2, CPT_SMALL), CHUNK), jnp.int32),
        pltpu.VMEM((2, CHUNK), jnp.int32),
        pltpu.VMEM((CHUNK, D), jnp.float32),
        pltpu.VMEM((CHUNK, D), jnp.float32),
        pltpu.VMEM_SHARED((ACC_ROWS, D), jnp.float32),
        pltpu.SemaphoreType.DMA,
        pltpu.SemaphoreType.DMA,
        pltpu.SemaphoreType.DMA,
        pltpu.SemaphoreType.DMA,
    ],
)


def _tc1_body(x_ref, w1_ref, degs_ref, h1s_ref, dcol_ref):
    deg = jnp.sum(degs_ref[...], axis=0)[:N_NODES] + 1.0
    d = lax.rsqrt(deg)[:, None]
    dcol_ref[...] = d
    h1s_ref[...] = jnp.dot(x_ref[...], w1_ref[...],
                           preferred_element_type=jnp.float32) * d


def _tc2_body(a_ref, h1s_ref, dcol_ref, b1_ref, w2_ref, h2s_ref):
    d = dcol_ref[...]
    t = (a_ref[0] + a_ref[1] + h1s_ref[...]) * d + b1_ref[...]
    h = jnp.maximum(t, 0.0)
    h2s_ref[...] = jnp.dot(h, w2_ref[...],
                           preferred_element_type=jnp.float32) * d


def _tc3_body(a_ref, h2s_ref, dcol_ref, b2_ref, out_ref):
    d = dcol_ref[...]
    out_ref[...] = (a_ref[0] + a_ref[1] + h2s_ref[...]) * d + b2_ref[...]


def kernel(x, edge_index, W1, b1, W2, b2):
    src = edge_index[0].astype(jnp.int32)
    dst = edge_index[1].astype(jnp.int32)
    pad = EDGES_PAD - src.shape[0]
    src2d = jnp.concatenate(
        [src, jnp.zeros((pad,), jnp.int32)]).reshape(NW * CPT, CHUNK)
    dst2d = jnp.concatenate(
        [dst, jnp.full((pad,), SAC_ROW, jnp.int32)]).reshape(NW * CPT, CHUNK)
    b1r = b1.reshape(1, D)
    b2r = b2.reshape(1, D)

    degs = _deg_call(dst2d)

    f32 = jnp.float32
    h1s, dcol = pl.pallas_call(
        _tc1_body,
        out_shape=(jax.ShapeDtypeStruct((N_NODES, D), f32),
                   jax.ShapeDtypeStruct((N_NODES, 1), f32)),
    )(x, W1, degs)

    agg1 = _agg_call(h1s, src2d, dst2d)

    h2s = pl.pallas_call(
        _tc2_body,
        out_shape=jax.ShapeDtypeStruct((N_NODES, D), f32),
    )(agg1, h1s, dcol, b1r, W2)

    agg2 = _agg_call(h2s, src2d, dst2d)

    out = pl.pallas_call(
        _tc3_body,
        out_shape=jax.ShapeDtypeStruct((N_NODES, D), f32),
    )(agg2, h2s, dcol, b2r)

    return out


# speedup vs baseline: 1.3558x; 1.3558x over previous
import jax
import jax.numpy as jnp
from jax import lax
from jax.experimental import pallas as pl
from jax.experimental.pallas import tpu as pltpu
from jax.experimental.pallas import tpu_sc as plsc

N_NODES = 10000
D = 128
NC, NS, L = 2, 16, 16
NW = NC * NS
CHUNK = 128
CPT = 80
CPT_BIG = 128
CPT_SMALL = 32
CPP = CPT_BIG + CPT_SMALL
BIG_CORE = 1
EDGES_PAD = NW * CPT * CHUNK
ACC_ROWS = 10112
ROWS_PT = ACC_ROWS // NS
LAST_ROWS = N_NODES - (NS - 1) * ROWS_PT
SAC_ROW = N_NODES


def _zero_vmem(ref, nrows, ncols):
    zv = jnp.zeros((L,), jnp.float32)

    def body(i, carry):
        r = i // (ncols // L)
        c = lax.rem(i, ncols // L)
        ref[r, pl.ds(c * L, L)] = zv
        return carry

    lax.fori_loop(0, nrows * (ncols // L), body, 0)


def _sc_deg_body(dst_hbm, out_hbm, dstb, hist):
    c = lax.axis_index("c")
    s = lax.axis_index("s")
    w = c * NS + s

    pltpu.sync_copy(dst_hbm.at[pl.ds(w * CPT, CPT)], dstb)

    zv = jnp.zeros((L,), jnp.float32)

    def z(i, carry):
        hist[pl.ds(i * L, L)] = zv
        return carry

    lax.fori_loop(0, ACC_ROWS // L, z, 0)

    ov = jnp.ones((L,), jnp.float32)

    def g(i, carry):
        j = i // (CHUNK // L)
        gidx = lax.rem(i, CHUNK // L)
        idx = dstb[j, pl.ds(gidx * L, L)]
        plsc.addupdate_scatter(hist, [idx], ov)
        return carry

    lax.fori_loop(0, CPT * (CHUNK // L), g, 0)

    pltpu.sync_copy(hist, out_hbm.at[w])


_deg_call = pl.kernel(
    _sc_deg_body,
    out_type=jax.ShapeDtypeStruct((NW, ACC_ROWS), jnp.float32),
    mesh=plsc.VectorSubcoreMesh(core_axis_name="c", subcore_axis_name="s"),
    compiler_params=pltpu.CompilerParams(needs_layout_passes=False),
    scratch_types=[
        pltpu.VMEM((CPT, CHUNK), jnp.int32),
        pltpu.VMEM((ACC_ROWS,), jnp.float32),
    ],
)


def _sc_agg_body(h_hbm, src_hbm, dst_hbm, out_hbm, srcb, dring, rows0, rows1,
                 accum, sem0, sem1, semd0, semd1):
    c = lax.axis_index("c")
    s = lax.axis_index("s")

    _zero_vmem(rows0, CHUNK, D)
    for k in range(ROWS_PT // CHUNK):
        pltpu.sync_copy(rows0, accum.at[pl.ds(s * ROWS_PT + k * CHUNK, CHUNK)])
    _rem = ROWS_PT % CHUNK
    if _rem:
        pltpu.sync_copy(
            rows0.at[pl.ds(0, _rem)],
            accum.at[pl.ds(s * ROWS_PT + (ROWS_PT // CHUNK) * CHUNK, _rem)])
    plsc.subcore_barrier()

    def run_chunks(base_row, n_chunks):
        pltpu.sync_copy(src_hbm.at[pl.ds(base_row, n_chunks)],
                        srcb.at[pl.ds(0, n_chunks)])
        pltpu.async_copy(dst_hbm.at[base_row], dring.at[0], semd0)
        pltpu.async_copy(h_hbm.at[srcb.at[0]], rows0, sem0)

        def pair(i, carry):
            j0 = 2 * i
            pltpu.async_copy(dst_hbm.at[base_row + j0 + 1], dring.at[1], semd1)
            pltpu.async_copy(h_hbm.at[srcb.at[j0 + 1]], rows1, sem1)

            pltpu.make_async_copy(dst_hbm.at[base_row + j0], dring.at[0], semd0).wait()
            pltpu.make_async_copy(h_hbm.at[srcb.at[j0]], rows0, sem0).wait()
            pltpu.sync_copy(rows0, accum.at[dring.at[0]], add=True)

            @pl.when(i < n_chunks // 2 - 1)
            def _():
                pltpu.async_copy(dst_hbm.at[base_row + j0 + 2], dring.at[0], semd0)
                pltpu.async_copy(h_hbm.at[srcb.at[j0 + 2]], rows0, sem0)

            pltpu.make_async_copy(dst_hbm.at[base_row + j0 + 1], dring.at[1], semd1).wait()
            pltpu.make_async_copy(h_hbm.at[srcb.at[j0 + 1]], rows1, sem1).wait()
            pltpu.sync_copy(rows1, accum.at[dring.at[1]], add=True)
            return carry

        lax.fori_loop(0, n_chunks // 2, pair, 0)

    @pl.when(c == BIG_CORE)
    def _():
        run_chunks(s * CPP, CPT_BIG)

    @pl.when(c != BIG_CORE)
    def _():
        run_chunks(s * CPP + CPT_BIG, CPT_SMALL)

    plsc.subcore_barrier()

    @pl.when(s < NS - 1)
    def _():
        pltpu.sync_copy(accum.at[pl.ds(s * ROWS_PT, ROWS_PT)],
                        out_hbm.at[c, pl.ds(s * ROWS_PT, ROWS_PT)])

    @pl.when(s == NS - 1)
    def _():
        pltpu.sync_copy(accum.at[pl.ds(s * ROWS_PT, LAST_ROWS)],
                        out_hbm.at[c, pl.ds(s * ROWS_PT, LAST_ROWS)])


_agg_call = pl.kernel(
    _sc_agg_body,
    out_type=jax.ShapeDtypeStruct((NC, N_NODES, D), jnp.float32),
    mesh=plsc.VectorSubcoreMesh(core_axis_name="c", subcore_axis_name="s"),
    scratch_types=[
        pltpu.VMEM((CPT_BIG, CHUNK), jnp.int32),
        pltpu.VMEM((2, CHUNK), jnp.int32),
        pltpu.VMEM((CHUNK, D), jnp.float32),
        pltpu.VMEM((CHUNK, D), jnp.float32),
        pltpu.VMEM_SHARED((ACC_ROWS, D), jnp.float32),
        pltpu.SemaphoreType.DMA,
        pltpu.SemaphoreType.DMA,
        pltpu.SemaphoreType.DMA,
        pltpu.SemaphoreType.DMA,
    ],
)


def _tc1_body(x_ref, w1_ref, degs_ref, h1s_ref, dcol_ref):
    deg = jnp.sum(degs_ref[...], axis=0)[:N_NODES] + 1.0
    d = lax.rsqrt(deg)[:, None]
    dcol_ref[...] = d
    h1s_ref[...] = jnp.dot(x_ref[...], w1_ref[...],
                           preferred_element_type=jnp.float32) * d


def _tc2_body(a_ref, h1s_ref, dcol_ref, b1_ref, w2_ref, h2s_ref):
    d = dcol_ref[...]
    t = (a_ref[0] + a_ref[1] + h1s_ref[...]) * d + b1_ref[...]
    h = jnp.maximum(t, 0.0)
    h2s_ref[...] = jnp.dot(h, w2_ref[...],
                           preferred_element_type=jnp.float32) * d


def _tc3_body(a_ref, h2s_ref, dcol_ref, b2_ref, out_ref):
    d = dcol_ref[...]
    out_ref[...] = (a_ref[0] + a_ref[1] + h2s_ref[...]) * d + b2_ref[...]


def kernel(x, edge_index, W1, b1, W2, b2):
    src = edge_index[0].astype(jnp.int32)
    dst = edge_index[1].astype(jnp.int32)
    pad = EDGES_PAD - src.shape[0]
    src2d = jnp.concatenate(
        [src, jnp.zeros((pad,), jnp.int32)]).reshape(NW * CPT, CHUNK)
    dst2d = jnp.concatenate(
        [dst, jnp.full((pad,), SAC_ROW, jnp.int32)]).reshape(NW * CPT, CHUNK)
    b1r = b1.reshape(1, D)
    b2r = b2.reshape(1, D)

    degs = _deg_call(dst2d)

    f32 = jnp.float32
    h1s, dcol = pl.pallas_call(
        _tc1_body,
        out_shape=(jax.ShapeDtypeStruct((N_NODES, D), f32),
                   jax.ShapeDtypeStruct((N_NODES, 1), f32)),
    )(x, W1, degs)

    agg1 = _agg_call(h1s, src2d, dst2d)

    h2s = pl.pallas_call(
        _tc2_body,
        out_shape=jax.ShapeDtypeStruct((N_NODES, D), f32),
    )(agg1, h1s, dcol, b1r, W2)

    agg2 = _agg_call(h2s, src2d, dst2d)

    out = pl.pallas_call(
        _tc3_body,
        out_shape=jax.ShapeDtypeStruct((N_NODES, D), f32),
    )(agg2, h2s, dcol, b2r)

    return out


# speedup vs baseline: 1.3608x; 1.0037x over previous
import jax
import jax.numpy as jnp
from jax import lax
from jax.experimental import pallas as pl
from jax.experimental.pallas import tpu as pltpu
from jax.experimental.pallas import tpu_sc as plsc

N_NODES = 10000
D = 128
NC, NS, L = 2, 16, 16
NW = NC * NS
CHUNK = 128
CPT = 80
CPT_BIG = 128
CPT_SMALL = 32
CPP = CPT_BIG + CPT_SMALL
BIG_CORE = 1
EDGES_PAD = NW * CPT * CHUNK
ACC_ROWS = 10112
ROWS_PT = ACC_ROWS // NS
LAST_ROWS = N_NODES - (NS - 1) * ROWS_PT
SAC_ROW = N_NODES


def _zero_vmem(ref, nrows, ncols):
    zv = jnp.zeros((L,), jnp.float32)

    def body(i, carry):
        r = i // (ncols // L)
        c = lax.rem(i, ncols // L)
        ref[r, pl.ds(c * L, L)] = zv
        return carry

    lax.fori_loop(0, nrows * (ncols // L), body, 0)


def _sc_deg_body(dst_hbm, out_hbm, dstb, hist):
    c = lax.axis_index("c")
    s = lax.axis_index("s")
    w = c * NS + s

    pltpu.sync_copy(dst_hbm.at[pl.ds(w * CPT, CPT)], dstb)

    zv = jnp.zeros((L,), jnp.float32)

    def z(i, carry):
        hist[pl.ds(i * L, L)] = zv
        return carry

    lax.fori_loop(0, ACC_ROWS // L, z, 0)

    ov = jnp.ones((L,), jnp.float32)

    def g(i, carry):
        j = i // (CHUNK // L)
        gidx = lax.rem(i, CHUNK // L)
        idx = dstb[j, pl.ds(gidx * L, L)]
        plsc.addupdate_scatter(hist, [idx], ov)
        return carry

    lax.fori_loop(0, CPT * (CHUNK // L), g, 0)

    pltpu.sync_copy(hist, out_hbm.at[w])


_deg_call = pl.kernel(
    _sc_deg_body,
    out_type=jax.ShapeDtypeStruct((NW, ACC_ROWS), jnp.float32),
    mesh=plsc.VectorSubcoreMesh(core_axis_name="c", subcore_axis_name="s"),
    compiler_params=pltpu.CompilerParams(needs_layout_passes=False),
    scratch_types=[
        pltpu.VMEM((CPT, CHUNK), jnp.int32),
        pltpu.VMEM((ACC_ROWS,), jnp.float32),
    ],
)


def _sc_agg_body(h_hbm, src_hbm, dst_hbm, out_hbm, srcb, dring, rows0, rows1,
                 accum, sem0, sem1, semd0, semd1):
    c = lax.axis_index("c")
    s = lax.axis_index("s")

    _zero_vmem(rows0, CHUNK, D)
    for k in range(ROWS_PT // CHUNK):
        pltpu.sync_copy(rows0, accum.at[pl.ds(s * ROWS_PT + k * CHUNK, CHUNK)])
    _rem = ROWS_PT % CHUNK
    if _rem:
        pltpu.sync_copy(
            rows0.at[pl.ds(0, _rem)],
            accum.at[pl.ds(s * ROWS_PT + (ROWS_PT // CHUNK) * CHUNK, _rem)])
    plsc.subcore_barrier()

    def run_chunks(base_row, n_chunks):
        pltpu.sync_copy(src_hbm.at[pl.ds(base_row, n_chunks)],
                        srcb.at[pl.ds(0, n_chunks)])

        HC = CHUNK // 2

        def gather(j, rbuf, sem):
            pltpu.async_copy(h_hbm.at[srcb.at[j, pl.ds(0, HC)]],
                             rbuf.at[pl.ds(0, HC)], sem)
            pltpu.async_copy(h_hbm.at[srcb.at[j, pl.ds(HC, HC)]],
                             rbuf.at[pl.ds(HC, HC)], sem)

        def gwait(j, rbuf, sem):
            pltpu.make_async_copy(h_hbm.at[srcb.at[j, pl.ds(0, HC)]],
                                  rbuf.at[pl.ds(0, HC)], sem).wait()
            pltpu.make_async_copy(h_hbm.at[srcb.at[j, pl.ds(HC, HC)]],
                                  rbuf.at[pl.ds(HC, HC)], sem).wait()

        pltpu.async_copy(dst_hbm.at[base_row], dring.at[0], semd0)
        gather(0, rows0, sem0)

        def pair(i, carry):
            j0 = 2 * i
            pltpu.async_copy(dst_hbm.at[base_row + j0 + 1], dring.at[1], semd1)
            gather(j0 + 1, rows1, sem1)

            pltpu.make_async_copy(dst_hbm.at[base_row + j0], dring.at[0], semd0).wait()
            gwait(j0, rows0, sem0)
            pltpu.sync_copy(rows0, accum.at[dring.at[0]], add=True)

            @pl.when(i < n_chunks // 2 - 1)
            def _():
                pltpu.async_copy(dst_hbm.at[base_row + j0 + 2], dring.at[0], semd0)
                gather(j0 + 2, rows0, sem0)

            pltpu.make_async_copy(dst_hbm.at[base_row + j0 + 1], dring.at[1], semd1).wait()
            gwait(j0 + 1, rows1, sem1)
            pltpu.sync_copy(rows1, accum.at[dring.at[1]], add=True)
            return carry

        lax.fori_loop(0, n_chunks // 2, pair, 0)

    @pl.when(c == BIG_CORE)
    def _():
        run_chunks(s * CPP, CPT_BIG)

    @pl.when(c != BIG_CORE)
    def _():
        run_chunks(s * CPP + CPT_BIG, CPT_SMALL)

    plsc.subcore_barrier()

    @pl.when(s < NS - 1)
    def _():
        pltpu.sync_copy(accum.at[pl.ds(s * ROWS_PT, ROWS_PT)],
                        out_hbm.at[c, pl.ds(s * ROWS_PT, ROWS_PT)])

    @pl.when(s == NS - 1)
    def _():
        pltpu.sync_copy(accum.at[pl.ds(s * ROWS_PT, LAST_ROWS)],
                        out_hbm.at[c, pl.ds(s * ROWS_PT, LAST_ROWS)])


_agg_call = pl.kernel(
    _sc_agg_body,
    out_type=jax.ShapeDtypeStruct((NC, N_NODES, D), jnp.float32),
    mesh=plsc.VectorSubcoreMesh(core_axis_name="c", subcore_axis_name="s"),
    scratch_types=[
        pltpu.VMEM((CPT_BIG, CHUNK), jnp.int32),
        pltpu.VMEM((2, CHUNK), jnp.int32),
        pltpu.VMEM((CHUNK, D), jnp.float32),
        pltpu.VMEM((CHUNK, D), jnp.float32),
        pltpu.VMEM_SHARED((ACC_ROWS, D), jnp.float32),
        pltpu.SemaphoreType.DMA,
        pltpu.SemaphoreType.DMA,
        pltpu.SemaphoreType.DMA,
        pltpu.SemaphoreType.DMA,
    ],
)


def _tc1_body(x_ref, w1_ref, degs_ref, h1s_ref, dcol_ref):
    deg = jnp.sum(degs_ref[...], axis=0)[:N_NODES] + 1.0
    d = lax.rsqrt(deg)[:, None]
    dcol_ref[...] = d
    h1s_ref[...] = jnp.dot(x_ref[...], w1_ref[...],
                           preferred_element_type=jnp.float32) * d


def _tc2_body(a_ref, h1s_ref, dcol_ref, b1_ref, w2_ref, h2s_ref):
    d = dcol_ref[...]
    t = (a_ref[0] + a_ref[1] + h1s_ref[...]) * d + b1_ref[...]
    h = jnp.maximum(t, 0.0)
    h2s_ref[...] = jnp.dot(h, w2_ref[...],
                           preferred_element_type=jnp.float32) * d


def _tc3_body(a_ref, h2s_ref, dcol_ref, b2_ref, out_ref):
    d = dcol_ref[...]
    out_ref[...] = (a_ref[0] + a_ref[1] + h2s_ref[...]) * d + b2_ref[...]


def kernel(x, edge_index, W1, b1, W2, b2):
    src = edge_index[0].astype(jnp.int32)
    dst = edge_index[1].astype(jnp.int32)
    pad = EDGES_PAD - src.shape[0]
    src2d = jnp.concatenate(
        [src, jnp.zeros((pad,), jnp.int32)]).reshape(NW * CPT, CHUNK)
    dst2d = jnp.concatenate(
        [dst, jnp.full((pad,), SAC_ROW, jnp.int32)]).reshape(NW * CPT, CHUNK)
    b1r = b1.reshape(1, D)
    b2r = b2.reshape(1, D)

    degs = _deg_call(dst2d)

    f32 = jnp.float32
    h1s, dcol = pl.pallas_call(
        _tc1_body,
        out_shape=(jax.ShapeDtypeStruct((N_NODES, D), f32),
                   jax.ShapeDtypeStruct((N_NODES, 1), f32)),
    )(x, W1, degs)

    agg1 = _agg_call(h1s, src2d, dst2d)

    h2s = pl.pallas_call(
        _tc2_body,
        out_shape=jax.ShapeDtypeStruct((N_NODES, D), f32),
    )(agg1, h1s, dcol, b1r, W2)

    agg2 = _agg_call(h2s, src2d, dst2d)

    out = pl.pallas_call(
        _tc3_body,
        out_shape=jax.ShapeDtypeStruct((N_NODES, D), f32),
    )(agg2, h2s, dcol, b2r)

    return out
